# baseline (speedup 1.0000x reference)
import functools

import jax
import jax.numpy as jnp
from jax import lax
from jax.experimental import pallas as pl
from jax.experimental.pallas import tpu as pltpu
from jax.experimental.pallas import tpu_sc as plsc

_NUM_CORES = 2
_NUM_SUBCORES = 16
_LANES = 16
_NUM_WORKERS = _NUM_CORES * _NUM_SUBCORES
_UNROLL = 8


def _lane_histogram_sc(chars, zeros_slab, vocab):
    bsz, seq = chars.shape
    rows_per = bsz // _NUM_WORKERS
    row_words = _LANES * vocab
    mesh = plsc.VectorSubcoreMesh(core_axis_name="c", subcore_axis_name="s")

    @functools.partial(
        pl.kernel,
        mesh=mesh,
        out_type=jax.ShapeDtypeStruct((bsz * row_words,), jnp.float32),
        compiler_params=pltpu.CompilerParams(needs_layout_passes=False),
        scratch_types=[
            pltpu.VMEM((rows_per, seq), jnp.int32),
            pltpu.VMEM((rows_per * row_words,), jnp.float32),
        ],
    )
    def hist_kernel(chars_hbm, zeros_hbm, out_hbm, chars_v, hist_v):
        wid = lax.axis_index("s") * _NUM_CORES + lax.axis_index("c")
        base = wid * rows_per
        pltpu.sync_copy(chars_hbm.at[pl.ds(base, rows_per)], chars_v)
        pltpu.sync_copy(zeros_hbm, hist_v)

        lane_off = lax.iota(jnp.int32, _LANES) * vocab
        ones = jnp.ones((_LANES,), jnp.float32)
        chunks = seq // _LANES

        for r in range(rows_per):
            row_off = lane_off + (r * row_words)

            def scatter_body(k, _, r=r, row_off=row_off):
                for j in range(_UNROLL):
                    col = pl.ds((k * _UNROLL + j) * _LANES, _LANES)
                    plsc.addupdate_scatter(
                        hist_v, [row_off + chars_v[r, col]], ones)
                return 0
            lax.fori_loop(0, chunks // _UNROLL, scatter_body, 0)

        pltpu.sync_copy(hist_v, out_hbm.at[pl.ds(base * row_words,
                                                 rows_per * row_words)])

    return hist_kernel(chars, zeros_slab)


def _dense_tc(lane_counts, e_rep, pos_slice, w_t, bias_row, inv_len,
              block_rows):
    bsz, width = lane_counts.shape
    bulk = w_t.shape[1]
    dim = e_rep.shape[1]
    pos_len = pos_slice.shape[0]

    def dense_kernel(counts_ref, ce_ref, pos_ref, wt_ref, b_ref, out_ref):
        pooled = jnp.dot(counts_ref[...], ce_ref[...],
                         preferred_element_type=jnp.float32)
        pos_sum = jnp.sum(pos_ref[...], axis=0, keepdims=True)
        x = (pooled + pos_sum) * inv_len
        out_ref[...] = jnp.dot(x, wt_ref[...],
                               preferred_element_type=jnp.float32) + b_ref[...]

    grid = (bsz // block_rows,)
    return pl.pallas_call(
        dense_kernel,
        grid=grid,
        in_specs=[
            pl.BlockSpec((block_rows, width), lambda i: (i, 0)),
            pl.BlockSpec((width, dim), lambda i: (0, 0)),
            pl.BlockSpec((pos_len, dim), lambda i: (0, 0)),
            pl.BlockSpec((dim, bulk), lambda i: (0, 0)),
            pl.BlockSpec((1, bulk), lambda i: (0, 0)),
        ],
        out_specs=pl.BlockSpec((block_rows, bulk), lambda i: (i, 0)),
        out_shape=jax.ShapeDtypeStruct((bsz, bulk), jnp.float32),
    )(lane_counts, e_rep, pos_slice, w_t, bias_row)


def kernel(chars, char_embed, pos_embed, to_bulk_w, to_bulk_b):
    bsz, seq = chars.shape
    vocab, _ = char_embed.shape
    rows_per = bsz // _NUM_WORKERS
    zeros_slab = jnp.zeros((rows_per * _LANES * vocab,), jnp.float32)
    lane_counts = _lane_histogram_sc(chars, zeros_slab, vocab)
    lane_counts = lane_counts.reshape(bsz, _LANES * vocab)
    e_rep = jnp.tile(char_embed, (_LANES, 1))
    pos_slice = pos_embed[:seq]
    w_t = to_bulk_w.T
    bias_row = to_bulk_b.reshape(1, -1)
    return _dense_tc(lane_counts, e_rep, pos_slice, w_t, bias_row,
                     1.0 / seq, block_rows=bsz // 4)


# speedup vs baseline: 1.4901x; 1.4901x over previous
import functools

import jax
import jax.numpy as jnp
from jax import lax
from jax.experimental import pallas as pl
from jax.experimental.pallas import tpu as pltpu
from jax.experimental.pallas import tpu_sc as plsc

_NUM_CORES = 2
_NUM_SUBCORES = 16
_LANES = 16
_NUM_WORKERS = _NUM_CORES * _NUM_SUBCORES
_UNROLL = 8


def _histogram_sc(chars, zeros_slab, vocab):
    bsz, seq = chars.shape
    rows_per = bsz // _NUM_WORKERS
    mesh = plsc.VectorSubcoreMesh(core_axis_name="c", subcore_axis_name="s")

    @functools.partial(
        pl.kernel,
        mesh=mesh,
        out_type=jax.ShapeDtypeStruct((bsz * vocab,), jnp.float32),
        compiler_params=pltpu.CompilerParams(needs_layout_passes=False),
        scratch_types=[
            pltpu.VMEM((rows_per, seq), jnp.int32),
            pltpu.VMEM((rows_per * vocab,), jnp.float32),
        ],
    )
    def hist_kernel(chars_hbm, zeros_hbm, out_hbm, chars_v, hist_v):
        wid = lax.axis_index("s") * _NUM_CORES + lax.axis_index("c")
        base = wid * rows_per
        pltpu.sync_copy(chars_hbm.at[pl.ds(base, rows_per)], chars_v)
        pltpu.sync_copy(zeros_hbm, hist_v)

        ones = jnp.ones((_LANES,), jnp.float32)
        chunks = seq // _LANES

        for r in range(rows_per):
            row_off = jnp.full((_LANES,), r * vocab, jnp.int32)

            def scatter_body(k, _, r=r, row_off=row_off):
                for j in range(_UNROLL):
                    col = pl.ds((k * _UNROLL + j) * _LANES, _LANES)
                    plsc.addupdate_scatter(
                        hist_v, [row_off + chars_v[r, col]], ones)
                return 0
            lax.fori_loop(0, chunks // _UNROLL, scatter_body, 0)

        pltpu.sync_copy(hist_v, out_hbm.at[pl.ds(base * vocab,
                                                 rows_per * vocab)])

    return hist_kernel(chars, zeros_slab)


def _dense_tc(counts, char_embed, pos_slice, w_t, bias_row, inv_len):
    bsz = counts.shape[0]
    bulk = w_t.shape[1]

    def dense_kernel(counts_ref, ce_ref, pos_ref, wt_ref, b_ref, out_ref):
        pooled = jnp.dot(counts_ref[...], ce_ref[...],
                         preferred_element_type=jnp.float32)
        pos_sum = jnp.sum(pos_ref[...], axis=0, keepdims=True)
        x = (pooled + pos_sum) * inv_len
        out_ref[...] = jnp.dot(x, wt_ref[...],
                               preferred_element_type=jnp.float32) + b_ref[...]

    return pl.pallas_call(
        dense_kernel,
        out_shape=jax.ShapeDtypeStruct((bsz, bulk), jnp.float32),
    )(counts, char_embed, pos_slice, w_t, bias_row)


def kernel(chars, char_embed, pos_embed, to_bulk_w, to_bulk_b):
    bsz, seq = chars.shape
    vocab, _ = char_embed.shape
    rows_per = bsz // _NUM_WORKERS
    zeros_slab = jnp.zeros((rows_per * vocab,), jnp.float32)
    counts = _histogram_sc(chars, zeros_slab, vocab).reshape(bsz, vocab)
    pos_slice = pos_embed[:seq]
    w_t = to_bulk_w.T
    bias_row = to_bulk_b.reshape(1, -1)
    return _dense_tc(counts, char_embed, pos_slice, w_t, bias_row, 1.0 / seq)


# speedup vs baseline: 1.6264x; 1.0915x over previous
import functools

import jax
import jax.numpy as jnp
from jax import lax
from jax.experimental import pallas as pl
from jax.experimental.pallas import tpu as pltpu
from jax.experimental.pallas import tpu_sc as plsc

_NUM_CORES = 2
_NUM_SUBCORES = 16
_LANES = 16
_NUM_WORKERS = _NUM_CORES * _NUM_SUBCORES
_UNROLL = 8


def _histogram_sc(chars, vocab):
    bsz, seq = chars.shape
    rows_per = bsz // _NUM_WORKERS
    mesh = plsc.VectorSubcoreMesh(core_axis_name="c", subcore_axis_name="s")

    @functools.partial(
        pl.kernel,
        mesh=mesh,
        out_type=jax.ShapeDtypeStruct((bsz, vocab), jnp.float32),
        compiler_params=pltpu.CompilerParams(needs_layout_passes=False),
        scratch_types=[
            pltpu.VMEM((rows_per, seq), jnp.int32),
            pltpu.VMEM((vocab,), jnp.float32),
            pltpu.VMEM((rows_per, vocab), jnp.float32),
        ],
    )
    def hist_kernel(chars_hbm, out_hbm, chars_v, hist_v, counts_v):
        wid = lax.axis_index("s") * _NUM_CORES + lax.axis_index("c")
        base = wid * rows_per
        pltpu.sync_copy(chars_hbm.at[pl.ds(base, rows_per)], chars_v)

        ones = jnp.ones((_LANES,), jnp.float32)
        zeros = jnp.zeros((_LANES,), jnp.float32)
        chunks = seq // _LANES

        def zero_body(i, _):
            hist_v[pl.ds(i * _LANES, _LANES)] = zeros
            return 0
        lax.fori_loop(0, vocab // _LANES, zero_body, 0)

        for r in range(rows_per):
            def scatter_body(k, _, r=r):
                for j in range(_UNROLL):
                    col = pl.ds((k * _UNROLL + j) * _LANES, _LANES)
                    plsc.addupdate_scatter(hist_v, [chars_v[r, col]], ones)
                return 0
            lax.fori_loop(0, chunks // _UNROLL, scatter_body, 0)

            def drain_body(i, _, r=r):
                col = pl.ds(i * _LANES, _LANES)
                counts_v[r, col] = hist_v[col]
                hist_v[col] = zeros
                return 0
            lax.fori_loop(0, vocab // _LANES, drain_body, 0)

        pltpu.sync_copy(counts_v, out_hbm.at[pl.ds(base, rows_per)])

    return hist_kernel(chars)


def _dense_tc(counts, char_embed, pos_embed, w_t, bias_row, seq):
    bsz, vocab = counts.shape
    dim = char_embed.shape[1]
    bulk = w_t.shape[1]
    inv_len = 1.0 / seq

    def dense_kernel(counts_ref, ce_ref, pos_ref, wt_ref, b_ref, out_ref):
        pooled = jnp.dot(counts_ref[...], ce_ref[...],
                         preferred_element_type=jnp.float32)
        pos_sum = jnp.sum(pos_ref[...], axis=0, keepdims=True)
        x = (pooled + pos_sum) * inv_len
        out_ref[...] = jnp.dot(x, wt_ref[...],
                               preferred_element_type=jnp.float32) + b_ref[...]

    return pl.pallas_call(
        dense_kernel,
        grid=(1,),
        in_specs=[
            pl.BlockSpec((bsz, vocab), lambda i: (0, 0)),
            pl.BlockSpec((vocab, dim), lambda i: (0, 0)),
            pl.BlockSpec((seq, dim), lambda i: (0, 0)),
            pl.BlockSpec((dim, bulk), lambda i: (0, 0)),
            pl.BlockSpec((1, bulk), lambda i: (0, 0)),
        ],
        out_specs=pl.BlockSpec((bsz, bulk), lambda i: (0, 0)),
        out_shape=jax.ShapeDtypeStruct((bsz, bulk), jnp.float32),
    )(counts, char_embed, pos_embed, w_t, bias_row)


def kernel(chars, char_embed, pos_embed, to_bulk_w, to_bulk_b):
    bsz, seq = chars.shape
    vocab, _ = char_embed.shape
    counts = _histogram_sc(chars, vocab)
    w_t = to_bulk_w.T
    bias_row = to_bulk_b.reshape(1, -1)
    return _dense_tc(counts, char_embed, pos_embed, w_t, bias_row, seq)
